# Initial kernel scaffold; baseline (speedup 1.0000x reference)
import jax
import jax.numpy as jnp
from jax.experimental import pallas as pl


def kernel(values, boundaries, table):
    raise NotImplementedError("write your pallas kernel here")


# speedup vs baseline: 10.9996x; 10.9996x over previous
import jax
import jax.numpy as jnp
from jax.experimental import pallas as pl

_BN = 4096


def _body(v_ref, lo_ref, sinv_ref, d_ref, o_ref):
    v = v_ref[0]
    lo = lo_ref[...]
    sinv = sinv_ref[...]
    u = jnp.clip((v - lo) * sinv, 0.0, 1.0)
    o_ref[...] = jax.lax.dot_general(
        u, d_ref[...],
        dimension_numbers=(((0,), (0,)), ((), ())),
        preferred_element_type=jnp.float32,
        precision=jax.lax.Precision.HIGHEST,
    )


def kernel(values, boundaries, table):
    n = values.shape[0]
    nb = table.shape[0]
    d = jnp.concatenate([table[:1], table[1:] - table[:-1]], axis=0)
    lo = jnp.concatenate(
        [jnp.full((1,), -3e30, jnp.float32), boundaries])[:, None]
    seg = boundaries[1:] - boundaries[:-1]
    sinv = jnp.concatenate(
        [jnp.ones((1,), jnp.float32), 1.0 / seg,
         jnp.full((1,), 1e30, jnp.float32)])[:, None]

    g = n // _BN
    v2 = values.reshape(g, 1, _BN)
    return pl.pallas_call(
        _body,
        grid=(g,),
        in_specs=[
            pl.BlockSpec((1, 1, _BN), lambda i: (i, 0, 0)),
            pl.BlockSpec((nb, 1), lambda i: (0, 0)),
            pl.BlockSpec((nb, 1), lambda i: (0, 0)),
            pl.BlockSpec((nb, 64), lambda i: (0, 0)),
        ],
        out_specs=pl.BlockSpec((_BN, 64), lambda i: (i, 0)),
        out_shape=jax.ShapeDtypeStruct((n, 64), jnp.float32),
    )(v2, lo, sinv, d)
